# Initial kernel scaffold; baseline (speedup 1.0000x reference)
import jax
import jax.numpy as jnp
from jax.experimental import pallas as pl


def kernel(nfeatures, efeatures, edge_index, weights):
    raise NotImplementedError("write your pallas kernel here")


# speedup vs baseline: 2.5663x; 2.5663x over previous
import functools

import jax
import jax.numpy as jnp
from jax import lax
from jax.experimental import pallas as pl
from jax.experimental.pallas import tpu as pltpu
from jax.experimental.pallas import tpu_sc as plsc

N = 10000
E = 320000
L = 16
CH = 128
NCHUNK = E // CH
NC = 2
NS = 16
ROWS_PER_TILE = N // NS

_mesh = plsc.VectorSubcoreMesh(core_axis_name="c", subcore_axis_name="s")
_sc_params = pltpu.CompilerParams(use_tc_tiling_on_sc=False)


def _tile_start(w):
    return w * 78 + min(w, 4)


@functools.partial(
    pl.kernel,
    out_type=(jax.ShapeDtypeStruct((E, L), jnp.float32),
              jax.ShapeDtypeStruct((E, L), jnp.float32)),
    mesh=_mesh,
    compiler_params=_sc_params,
    scratch_types=[
        pltpu.VMEM((80, CH), jnp.int32),
        pltpu.VMEM((CH, L), jnp.float32),
        pltpu.SemaphoreType.DMA,
    ],
)
def _sc_gather(node_hbm, src_hbm, dst_hbm, gsrc_hbm, gdst_hbm,
               idx_v, rows_v, sem):
    c = lax.axis_index("c")
    s = lax.axis_index("s")
    wid = c * NS + s
    start = wid * 78 + jnp.minimum(wid, 4)
    for k in range(2):
        ih = src_hbm if k == 0 else dst_hbm
        oh = gsrc_hbm if k == 0 else gdst_hbm
        pltpu.sync_copy(ih.at[wid], idx_v)

        def _one(j):
            pltpu.async_copy(node_hbm.at[idx_v.at[j]], rows_v, sem).wait()
            pltpu.sync_copy(rows_v, oh.at[pl.ds((start + j) * CH, CH)])

        @pl.loop(0, 78)
        def _(j):
            _one(j)

        @pl.when(wid < 4)
        def _():
            _one(78)


@functools.partial(
    pl.kernel,
    out_type=jax.ShapeDtypeStruct((NC, N, L), jnp.float32),
    mesh=_mesh,
    compiler_params=_sc_params,
    scratch_types=[
        pltpu.VMEM((80, CH), jnp.int32),
        pltpu.VMEM((CH, L), jnp.float32),
        pltpu.VMEM_SHARED((N, L), jnp.float32),
        pltpu.SemaphoreType.DMA,
    ],
)
def _sc_scatter(edge_hbm, dst_hbm, zeros_hbm, out_hbm,
                idx_v, rows_v, acc, sem):
    c = lax.axis_index("c")
    s = lax.axis_index("s")
    wid = c * NS + s
    @pl.when(s < NS - 1)
    def _():
        pltpu.sync_copy(zeros_hbm.at[pl.ds(0, 624)],
                        acc.at[pl.ds(s * 624, 624)])

    @pl.when(s == NS - 1)
    def _():
        pltpu.sync_copy(zeros_hbm, acc.at[pl.ds(9360, 640)])

    plsc.subcore_barrier()

    start = wid * 78 + jnp.minimum(wid, 4)
    pltpu.sync_copy(dst_hbm.at[wid], idx_v)

    def _one(j):
        pltpu.sync_copy(edge_hbm.at[pl.ds((start + j) * CH, CH)], rows_v)
        pltpu.sync_copy(rows_v, acc.at[idx_v.at[j]], add=True)

    @pl.loop(0, 78)
    def _(j):
        _one(j)

    @pl.when(wid < 4)
    def _():
        _one(78)

    plsc.subcore_barrier()

    @pl.when(s < NS - 1)
    def _():
        pltpu.sync_copy(acc.at[pl.ds(s * 624, 624)],
                        out_hbm.at[c].at[pl.ds(s * 624, 624)])

    @pl.when(s == NS - 1)
    def _():
        pltpu.sync_copy(acc.at[pl.ds(9360, 640)],
                        out_hbm.at[c].at[pl.ds(9360, 640)])


def _leaky(x):
    return jnp.maximum(x, 0.01 * x)


def _ln(f, g, b):
    mu = jnp.mean(f, axis=-1, keepdims=True)
    d = f - mu
    var = jnp.mean(d * d, axis=-1, keepdims=True)
    return d * lax.rsqrt(var + 1e-5) * g + b


def _dot(a, b):
    return jnp.dot(a, b, preferred_element_type=jnp.float32)


def _full_specs(arrs, ngrid_dims=1):
    zeros = (0,) * 1

    def mk(a):
        nd = a.ndim
        return pl.BlockSpec(a.shape, lambda i, _nd=nd: (0,) * _nd)

    return [mk(a) for a in arrs]


def _flat_mlp(p):
    out = [p['in']['W'], p['in']['b'].reshape(1, -1),
           p['hid'][0]['W'], p['hid'][0]['b'].reshape(1, -1),
           p['out']['W'], p['out']['b'].reshape(1, -1)]
    if 'ln' in p:
        out += [p['ln']['g'].reshape(1, -1), p['ln']['b'].reshape(1, -1)]
    return out


def _enc_kernel(x_ref, w1, b1, w2, b2, w3, b3, g, bl, o_ref):
    f = _leaky(_dot(x_ref[...], w1[...]) + b1[...])
    f = _leaky(_dot(f, w2[...]) + b2[...])
    f = _dot(f, w3[...]) + b3[...]
    o_ref[...] = _ln(f, g[...], bl[...])


def _enc(x, p, block_rows):
    n, fin = x.shape
    ws = _flat_mlp(p)
    grid = (n // block_rows,)
    return pl.pallas_call(
        _enc_kernel,
        grid=grid,
        in_specs=[pl.BlockSpec((block_rows, fin), lambda i: (i, 0))]
                 + _full_specs(ws),
        out_specs=pl.BlockSpec((block_rows, L), lambda i: (i, 0)),
        out_shape=jax.ShapeDtypeStruct((n, L), jnp.float32),
    )(x, *ws)


def _edge_kernel(e_ref, s_ref, d_ref, w1e, w1s, w1d, b1, w2, b2, w3, b3,
                 g, bl, o_ref):
    x = e_ref[...]
    f = (_dot(x, w1e[...]) + _dot(s_ref[...], w1s[...])
         + _dot(d_ref[...], w1d[...]) + b1[...])
    f = _leaky(f)
    f = _leaky(_dot(f, w2[...]) + b2[...])
    f = _dot(f, w3[...]) + b3[...]
    o_ref[...] = _ln(f, g[...], bl[...]) + x


def _edge_mlp(edge, gsrc, gdst, p, block_rows=8000):
    w1, b1, w2, b2, w3, b3, g, bl = _flat_mlp(p)
    ws = [w1[:L], w1[L:2 * L], w1[2 * L:], b1, w2, b2, w3, b3, g, bl]
    grid = (E // block_rows,)
    espec = pl.BlockSpec((block_rows, L), lambda i: (i, 0))
    return pl.pallas_call(
        _edge_kernel,
        grid=grid,
        in_specs=[espec, espec, espec] + _full_specs(ws),
        out_specs=espec,
        out_shape=jax.ShapeDtypeStruct((E, L), jnp.float32),
    )(edge, gsrc, gdst, *ws)


def _node_kernel(n_ref, p_ref, w1a, w1b, b1, w2, b2, w3, b3, g, bl, o_ref):
    x = n_ref[...]
    ps = p_ref[0] + p_ref[1]
    f = _dot(x, w1a[...]) + _dot(ps, w1b[...]) + b1[...]
    f = _leaky(f)
    f = _leaky(_dot(f, w2[...]) + b2[...])
    f = _dot(f, w3[...]) + b3[...]
    o_ref[...] = _ln(f, g[...], bl[...]) + x


def _node_mlp(node, parts, p, block_rows=2000):
    w1, b1, w2, b2, w3, b3, g, bl = _flat_mlp(p)
    ws = [w1[:L], w1[L:], b1, w2, b2, w3, b3, g, bl]
    grid = (N // block_rows,)
    return pl.pallas_call(
        _node_kernel,
        grid=grid,
        in_specs=[pl.BlockSpec((block_rows, L), lambda i: (i, 0)),
                  pl.BlockSpec((NC, block_rows, L), lambda i: (0, i, 0))]
                 + _full_specs(ws),
        out_specs=pl.BlockSpec((block_rows, L), lambda i: (i, 0)),
        out_shape=jax.ShapeDtypeStruct((N, L), jnp.float32),
    )(node, parts, *ws)


def _dec_kernel(x_ref, w1, b1, w2, b2, w3, b3, o_ref):
    f = _leaky(_dot(x_ref[...], w1[...]) + b1[...])
    f = _leaky(_dot(f, w2[...]) + b2[...])
    o_ref[...] = _dot(f, w3[...]) + b3[...]


def _dec(x, p, block_rows=2000):
    ws = _flat_mlp(p)
    nout = ws[4].shape[1]
    grid = (N // block_rows,)
    return pl.pallas_call(
        _dec_kernel,
        grid=grid,
        in_specs=[pl.BlockSpec((block_rows, L), lambda i: (i, 0))]
                 + _full_specs(ws),
        out_specs=pl.BlockSpec((block_rows, nout), lambda i: (i, 0)),
        out_shape=jax.ShapeDtypeStruct((N, nout), jnp.float32),
    )(x, *ws)


def _tile_blocks(idx_flat):
    idx2d = idx_flat.reshape(NCHUNK, CH)
    p = jnp.concatenate([idx2d, jnp.zeros((80, CH), jnp.int32)], axis=0)
    return jnp.stack([p[_tile_start(w):_tile_start(w) + 80] for w in range(32)])


def kernel(nfeatures, efeatures, edge_index, weights):
    src3d = _tile_blocks(edge_index[0])
    dst3d = _tile_blocks(edge_index[1])
    zeros = jnp.zeros((640, L), jnp.float32)

    node = _enc(nfeatures, weights['enc_nodes'], block_rows=2000)
    edge = _enc(efeatures, weights['enc_edges'], block_rows=8000)

    for i in range(8):
        gsrc, gdst = _sc_gather(node, src3d, dst3d)
        edge = _edge_mlp(edge, gsrc, gdst, weights['proc_edges'][i])
        parts = _sc_scatter(edge, dst3d, zeros)
        node = _node_mlp(node, parts, weights['proc_nodes'][i])

    return _dec(node, weights['dec'])


# speedup vs baseline: 3.1082x; 1.2112x over previous
import functools

import jax
import jax.numpy as jnp
from jax import lax
from jax.experimental import pallas as pl
from jax.experimental.pallas import tpu as pltpu
from jax.experimental.pallas import tpu_sc as plsc

N = 10000
E = 320000
L = 16
CH = 128
NCHUNK = E // CH
NC = 2
NS = 16
ROWS_PER_TILE = N // NS

_mesh = plsc.VectorSubcoreMesh(core_axis_name="c", subcore_axis_name="s")
_sc_params = pltpu.CompilerParams(use_tc_tiling_on_sc=False)


def _tile_start(w):
    return w * 78 + min(w, 4)


G = 13
GB = G * CH
NGRP = 6


@functools.partial(
    pl.kernel,
    out_type=(jax.ShapeDtypeStruct((E, L), jnp.float32),
              jax.ShapeDtypeStruct((E, L), jnp.float32)),
    mesh=_mesh,
    compiler_params=_sc_params,
    scratch_types=[
        pltpu.VMEM((80, CH), jnp.int32),
        pltpu.VMEM((2 * GB, L), jnp.float32),
        pltpu.SemaphoreType.DMA,
        pltpu.SemaphoreType.DMA,
    ],
)
def _sc_gather(node_hbm, src_hbm, dst_hbm, gsrc_hbm, gdst_hbm,
               idx_v, rows_v, sem_g, sem_o):
    c = lax.axis_index("c")
    s = lax.axis_index("s")
    wid = c * NS + s
    start = wid * 78 + jnp.minimum(wid, 4)
    for k in range(2):
        ih = src_hbm if k == 0 else dst_hbm
        oh = gsrc_hbm if k == 0 else gdst_hbm
        pltpu.sync_copy(ih.at[wid], idx_v)

        @pl.loop(0, NGRP)
        def _(g):
            b = (g % 2) * GB

            @pl.when(g >= 2)
            def _():
                pltpu.make_async_copy(oh.at[pl.ds(0, GB)],
                                      rows_v.at[pl.ds(0, GB)], sem_o).wait()

            @pl.loop(0, G)
            def _(j):
                pltpu.async_copy(node_hbm.at[idx_v.at[g * G + j]],
                                 rows_v.at[pl.ds(b + j * CH, CH)], sem_g)

            @pl.loop(0, G)
            def _(j):
                pltpu.make_async_copy(node_hbm.at[pl.ds(0, CH)],
                                      rows_v.at[pl.ds(0, CH)], sem_g).wait()

            pltpu.async_copy(rows_v.at[pl.ds(b, GB)],
                             oh.at[pl.ds((start + g * G) * CH, GB)], sem_o)

        for _ in range(2):
            pltpu.make_async_copy(oh.at[pl.ds(0, GB)],
                                  rows_v.at[pl.ds(0, GB)], sem_o).wait()

        @pl.when(wid < 4)
        def _():
            pltpu.async_copy(node_hbm.at[idx_v.at[78]],
                             rows_v.at[pl.ds(0, CH)], sem_g).wait()
            pltpu.sync_copy(rows_v.at[pl.ds(0, CH)],
                            oh.at[pl.ds((start + 78) * CH, CH)])


@functools.partial(
    pl.kernel,
    out_type=jax.ShapeDtypeStruct((NC, N, L), jnp.float32),
    mesh=_mesh,
    compiler_params=_sc_params,
    scratch_types=[
        pltpu.VMEM((80, CH), jnp.int32),
        pltpu.VMEM((2 * GB, L), jnp.float32),
        pltpu.VMEM_SHARED((N, L), jnp.float32),
        pltpu.SemaphoreType.DMA,
        pltpu.SemaphoreType.DMA,
    ],
)
def _sc_scatter(edge_hbm, dst_hbm, zeros_hbm, out_hbm,
                idx_v, rows_v, acc, sem_l, sem_s):
    c = lax.axis_index("c")
    s = lax.axis_index("s")
    wid = c * NS + s
    @pl.when(s < NS - 1)
    def _():
        pltpu.sync_copy(zeros_hbm.at[pl.ds(0, 624)],
                        acc.at[pl.ds(s * 624, 624)])

    @pl.when(s == NS - 1)
    def _():
        pltpu.sync_copy(zeros_hbm, acc.at[pl.ds(9360, 640)])

    plsc.subcore_barrier()

    start = wid * 78 + jnp.minimum(wid, 4)
    pltpu.sync_copy(dst_hbm.at[wid], idx_v)

    pltpu.async_copy(edge_hbm.at[pl.ds(start * CH, GB)],
                     rows_v.at[pl.ds(0, GB)], sem_l)

    @pl.loop(0, NGRP)
    def _(g):
        b = (g % 2) * GB
        pltpu.make_async_copy(edge_hbm.at[pl.ds(0, GB)],
                              rows_v.at[pl.ds(0, GB)], sem_l).wait()

        @pl.when(g >= 1)
        def _():
            @pl.loop(0, G)
            def _(j):
                pltpu.make_async_copy(edge_hbm.at[pl.ds(0, CH)],
                                      rows_v.at[pl.ds(0, CH)], sem_s).wait()

        @pl.when(g < NGRP - 1)
        def _():
            pltpu.async_copy(edge_hbm.at[pl.ds((start + (g + 1) * G) * CH, GB)],
                             rows_v.at[pl.ds(GB - b, GB)], sem_l)

        @pl.loop(0, G)
        def _(j):
            pltpu.async_copy(rows_v.at[pl.ds(b + j * CH, CH)],
                             acc.at[idx_v.at[g * G + j]], sem_s, add=True)

    @pl.loop(0, G)
    def _(j):
        pltpu.make_async_copy(edge_hbm.at[pl.ds(0, CH)],
                              rows_v.at[pl.ds(0, CH)], sem_s).wait()

    @pl.when(wid < 4)
    def _():
        pltpu.sync_copy(edge_hbm.at[pl.ds((start + 78) * CH, CH)],
                        rows_v.at[pl.ds(0, CH)])
        pltpu.sync_copy(rows_v.at[pl.ds(0, CH)], acc.at[idx_v.at[78]], add=True)

    plsc.subcore_barrier()

    @pl.when(s < NS - 1)
    def _():
        pltpu.sync_copy(acc.at[pl.ds(s * 624, 624)],
                        out_hbm.at[c].at[pl.ds(s * 624, 624)])

    @pl.when(s == NS - 1)
    def _():
        pltpu.sync_copy(acc.at[pl.ds(9360, 640)],
                        out_hbm.at[c].at[pl.ds(9360, 640)])


def _leaky(x):
    return jnp.maximum(x, 0.01 * x)


def _ln(f, g, b):
    mu = jnp.mean(f, axis=-1, keepdims=True)
    d = f - mu
    var = jnp.mean(d * d, axis=-1, keepdims=True)
    return d * lax.rsqrt(var + 1e-5) * g + b


def _dot(a, b):
    return jnp.dot(a, b, preferred_element_type=jnp.float32)


def _full_specs(arrs, ngrid_dims=1):
    zeros = (0,) * 1

    def mk(a):
        nd = a.ndim
        return pl.BlockSpec(a.shape, lambda i, _nd=nd: (0,) * _nd)

    return [mk(a) for a in arrs]


def _flat_mlp(p):
    out = [p['in']['W'], p['in']['b'].reshape(1, -1),
           p['hid'][0]['W'], p['hid'][0]['b'].reshape(1, -1),
           p['out']['W'], p['out']['b'].reshape(1, -1)]
    if 'ln' in p:
        out += [p['ln']['g'].reshape(1, -1), p['ln']['b'].reshape(1, -1)]
    return out


def _enc_kernel(x_ref, w1, b1, w2, b2, w3, b3, g, bl, o_ref):
    f = _leaky(_dot(x_ref[...], w1[...]) + b1[...])
    f = _leaky(_dot(f, w2[...]) + b2[...])
    f = _dot(f, w3[...]) + b3[...]
    o_ref[...] = _ln(f, g[...], bl[...])


def _enc(x, p, block_rows):
    n, fin = x.shape
    ws = _flat_mlp(p)
    grid = (n // block_rows,)
    return pl.pallas_call(
        _enc_kernel,
        grid=grid,
        in_specs=[pl.BlockSpec((block_rows, fin), lambda i: (i, 0))]
                 + _full_specs(ws),
        out_specs=pl.BlockSpec((block_rows, L), lambda i: (i, 0)),
        out_shape=jax.ShapeDtypeStruct((n, L), jnp.float32),
    )(x, *ws)


def _edge_kernel(e_ref, s_ref, d_ref, w1e, w1s, w1d, b1, w2, b2, w3, b3,
                 g, bl, o_ref):
    x = e_ref[...]
    f = (_dot(x, w1e[...]) + _dot(s_ref[...], w1s[...])
         + _dot(d_ref[...], w1d[...]) + b1[...])
    f = _leaky(f)
    f = _leaky(_dot(f, w2[...]) + b2[...])
    f = _dot(f, w3[...]) + b3[...]
    o_ref[...] = _ln(f, g[...], bl[...]) + x


def _edge_mlp(edge, gsrc, gdst, p, block_rows=8000):
    w1, b1, w2, b2, w3, b3, g, bl = _flat_mlp(p)
    ws = [w1[:L], w1[L:2 * L], w1[2 * L:], b1, w2, b2, w3, b3, g, bl]
    grid = (E // block_rows,)
    espec = pl.BlockSpec((block_rows, L), lambda i: (i, 0))
    return pl.pallas_call(
        _edge_kernel,
        grid=grid,
        in_specs=[espec, espec, espec] + _full_specs(ws),
        out_specs=espec,
        out_shape=jax.ShapeDtypeStruct((E, L), jnp.float32),
    )(edge, gsrc, gdst, *ws)


def _node_kernel(n_ref, p_ref, w1a, w1b, b1, w2, b2, w3, b3, g, bl, o_ref):
    x = n_ref[...]
    ps = p_ref[0] + p_ref[1]
    f = _dot(x, w1a[...]) + _dot(ps, w1b[...]) + b1[...]
    f = _leaky(f)
    f = _leaky(_dot(f, w2[...]) + b2[...])
    f = _dot(f, w3[...]) + b3[...]
    o_ref[...] = _ln(f, g[...], bl[...]) + x


def _node_mlp(node, parts, p, block_rows=2000):
    w1, b1, w2, b2, w3, b3, g, bl = _flat_mlp(p)
    ws = [w1[:L], w1[L:], b1, w2, b2, w3, b3, g, bl]
    grid = (N // block_rows,)
    return pl.pallas_call(
        _node_kernel,
        grid=grid,
        in_specs=[pl.BlockSpec((block_rows, L), lambda i: (i, 0)),
                  pl.BlockSpec((NC, block_rows, L), lambda i: (0, i, 0))]
                 + _full_specs(ws),
        out_specs=pl.BlockSpec((block_rows, L), lambda i: (i, 0)),
        out_shape=jax.ShapeDtypeStruct((N, L), jnp.float32),
    )(node, parts, *ws)


def _dec_kernel(x_ref, w1, b1, w2, b2, w3, b3, o_ref):
    f = _leaky(_dot(x_ref[...], w1[...]) + b1[...])
    f = _leaky(_dot(f, w2[...]) + b2[...])
    o_ref[...] = _dot(f, w3[...]) + b3[...]


def _dec(x, p, block_rows=2000):
    ws = _flat_mlp(p)
    nout = ws[4].shape[1]
    grid = (N // block_rows,)
    return pl.pallas_call(
        _dec_kernel,
        grid=grid,
        in_specs=[pl.BlockSpec((block_rows, L), lambda i: (i, 0))]
                 + _full_specs(ws),
        out_specs=pl.BlockSpec((block_rows, nout), lambda i: (i, 0)),
        out_shape=jax.ShapeDtypeStruct((N, nout), jnp.float32),
    )(x, *ws)


def _tile_blocks(idx_flat):
    idx2d = idx_flat.reshape(NCHUNK, CH)
    p = jnp.concatenate([idx2d, jnp.zeros((80, CH), jnp.int32)], axis=0)
    return jnp.stack([p[_tile_start(w):_tile_start(w) + 80] for w in range(32)])


def kernel(nfeatures, efeatures, edge_index, weights):
    src3d = _tile_blocks(edge_index[0])
    dst3d = _tile_blocks(edge_index[1])
    zeros = jnp.zeros((640, L), jnp.float32)

    node = _enc(nfeatures, weights['enc_nodes'], block_rows=2000)
    edge = _enc(efeatures, weights['enc_edges'], block_rows=8000)

    for i in range(8):
        gsrc, gdst = _sc_gather(node, src3d, dst3d)
        edge = _edge_mlp(edge, gsrc, gdst, weights['proc_edges'][i])
        parts = _sc_scatter(edge, dst3d, zeros)
        node = _node_mlp(node, parts, weights['proc_nodes'][i])

    return _dec(node, weights['dec'])
